# Initial kernel scaffold; baseline (speedup 1.0000x reference)
import functools

import jax
import jax.numpy as jnp
from jax import lax
from jax.experimental import pallas as pl
from jax.experimental.pallas import tpu as pltpu
from jax.experimental.pallas import tpu_sc as plsc

B, N, M, D, KNB = 4, 2048, 512, 128, 6
NM = N + M
TOT = B * NM
D2 = 2 * D

_SC = plsc.get_sparse_core_info()
_NC, _NS = _SC.num_cores, _SC.num_subcores
_NW = _NC * _NS
_NPW = TOT // _NW
_CH = 16
_NCHUNK = _NPW // _CH


def _encode_body(x_ref, ws_ref, bs_ref, wt_ref, bt_ref, we_ref, be_ref,
                 wn_ref, a_ref, bc_ref, fp_ref):
    nb = pl.program_id(1)
    is_src = nb < (N // 512)
    w = jnp.where(is_src, ws_ref[...], wt_ref[...])
    bias = jnp.where(is_src, bs_ref[...], bt_ref[...])
    x = x_ref[0]
    emb = (x[:, 0][:, None] * w[0][None, :]
           + x[:, 1][:, None] * w[1][None, :]
           + x[:, 2][:, None] * w[2][None, :])
    emb = jnp.maximum(emb + bias, 0.0)
    a_ref[0] = jnp.dot(emb, we_ref[:D, :],
                       preferred_element_type=jnp.float32)
    bc_ref[0] = (jnp.dot(emb, we_ref[D:, :],
                         preferred_element_type=jnp.float32)
                 + be_ref[...])
    fp_ref[0] = jnp.dot(emb, wn_ref[:D, :],
                        preferred_element_type=jnp.float32)


def _encode(xyz, W_src, b_src, W_tgt, b_tgt, W_edge, b_edge, W_node):
    nblk = NM // 512
    full = lambda shape: pl.BlockSpec(shape, lambda b, n: (0,) * len(shape))
    return pl.pallas_call(
        _encode_body,
        grid=(B, nblk),
        in_specs=[
            pl.BlockSpec((1, 512, 3), lambda b, n: (b, n, 0)),
            full((3, D)), full((1, D)), full((3, D)), full((1, D)),
            full((D2, D2)), full((1, D2)), full((3 * D, D)),
        ],
        out_specs=[
            pl.BlockSpec((1, 512, D2), lambda b, n: (b, n, 0)),
            pl.BlockSpec((1, 512, D2), lambda b, n: (b, n, 0)),
            pl.BlockSpec((1, 512, D), lambda b, n: (b, n, 0)),
        ],
        out_shape=[
            jax.ShapeDtypeStruct((B, NM, D2), jnp.float32),
            jax.ShapeDtypeStruct((B, NM, D2), jnp.float32),
            jax.ShapeDtypeStruct((B, NM, D), jnp.float32),
        ],
    )(xyz, W_src, b_src.reshape(1, D), W_tgt, b_tgt.reshape(1, D),
      W_edge, b_edge.reshape(1, D2), W_node)


def _knn_body(xr_ref, xt_ref, out_ref, *, rows, cols, goff):
    b = pl.program_id(0)
    rb = pl.program_id(1)
    xr = xr_ref[0]
    xt = xt_ref[0]
    d = None
    for c in range(3):
        diff = xr[:, c][:, None] - xt[c][None, :]
        sq = diff * diff
        d = sq if d is None else d + sq
    colf = lax.broadcasted_iota(jnp.float32, (rows, cols), 1)
    rowf = (lax.broadcasted_iota(jnp.float32, (rows, cols), 0)
            + (rb * rows).astype(jnp.float32))
    d = jnp.where(colf == rowf, jnp.float32(1e10), d)
    sent = jnp.float32(2 * cols)
    big = jnp.float32(3e38)
    js = []
    for k in range(KNB):
        m = jnp.min(d, axis=1, keepdims=True)
        cand = jnp.where(d == m, colf, sent)
        j = jnp.min(cand, axis=1, keepdims=True)
        js.append(j)
        if k < KNB - 1:
            d = jnp.where(cand == j, big, d)
    idx = jnp.concatenate(js, axis=1).astype(jnp.int32)
    out_ref[0] = idx + (b * NM + goff)


def _knn(xyz, xyzT, rows, goff):
    npts = xyz.shape[1]
    nblk = npts // rows
    body = functools.partial(_knn_body, rows=rows, cols=npts, goff=goff)
    return pl.pallas_call(
        body,
        grid=(B, nblk),
        in_specs=[
            pl.BlockSpec((1, rows, 3), lambda b, n: (b, n, 0)),
            pl.BlockSpec((1, 3, npts), lambda b, n: (b, 0, 0)),
        ],
        out_specs=pl.BlockSpec((1, rows, KNB), lambda b, n: (b, n, 0)),
        out_shape=jax.ShapeDtypeStruct((B, npts, KNB), jnp.int32),
    )(xyz, xyzT)


def _sc_agg_body(a_hbm, bc_hbm, idx_hbm, out_hbm, idx_v, rows_v, bc_v,
                 acc_v, sem):
    wid = lax.axis_index("s") * _NC + lax.axis_index("c")
    base0 = wid * _NPW

    def chunk_body(ch, carry):
        nb = base0 + ch * _CH
        pltpu.sync_copy(idx_hbm.at[pl.ds(nb * KNB, _CH * KNB)], idx_v)
        cp = pltpu.async_copy(a_hbm.at[idx_v], rows_v, sem)
        pltpu.sync_copy(bc_hbm.at[pl.ds(nb, _CH)], bc_v)
        cp.wait()

        def node_body(i, c2):
            for c in range(D2 // 16):
                sl = pl.ds(c * 16, 16)
                bcv = bc_v[i, sl]
                acc = jnp.maximum(rows_v[i * KNB, sl] + bcv, 0.0)
                for k in range(1, KNB):
                    acc = acc + jnp.maximum(rows_v[i * KNB + k, sl] + bcv,
                                            0.0)
                acc_v[i, sl] = acc
            return c2

        lax.fori_loop(0, _CH, node_body, 0)
        pltpu.sync_copy(acc_v, out_hbm.at[pl.ds(nb, _CH)])
        return carry

    lax.fori_loop(0, _NCHUNK, chunk_body, 0)


def _sc_agg(a2, bc2, idxf):
    mesh = plsc.VectorSubcoreMesh(core_axis_name="c", subcore_axis_name="s")
    fn = pl.kernel(
        _sc_agg_body,
        out_type=jax.ShapeDtypeStruct((TOT, D2), jnp.float32),
        mesh=mesh,
        scratch_types=[
            pltpu.VMEM((_CH * KNB,), jnp.int32),
            pltpu.VMEM((_CH * KNB, D2), jnp.float32),
            pltpu.VMEM((_CH, D2), jnp.float32),
            pltpu.VMEM((_CH, D2), jnp.float32),
            pltpu.SemaphoreType.DMA,
        ],
    )
    return fn(a2, bc2, idxf)


def _final_body(fp_ref, agg_ref, wn_ref, bn_ref, out_ref):
    out_ref[...] = jnp.maximum(
        fp_ref[...]
        + jnp.dot(agg_ref[...], wn_ref[...],
                  preferred_element_type=jnp.float32)
        + bn_ref[...], 0.0)


def _final(fp2, agg, W_node, b_node):
    nblk = TOT // 512
    return pl.pallas_call(
        _final_body,
        grid=(nblk,),
        in_specs=[
            pl.BlockSpec((512, D), lambda n: (n, 0)),
            pl.BlockSpec((512, D2), lambda n: (n, 0)),
            pl.BlockSpec((D2, D), lambda n: (0, 0)),
            pl.BlockSpec((1, D), lambda n: (0, 0)),
        ],
        out_specs=pl.BlockSpec((512, D), lambda n: (n, 0)),
        out_shape=jax.ShapeDtypeStruct((TOT, D), jnp.float32),
    )(fp2, agg, W_node[D:, :], b_node.reshape(1, D))


def kernel(src, tgt, W_src, b_src, W_tgt, b_tgt, W_edge, b_edge, W_node,
           b_node):
    xyz = jnp.concatenate([src, tgt], axis=1)
    a, bc, fp = _encode(xyz, W_src, b_src, W_tgt, b_tgt, W_edge, b_edge,
                        W_node)
    idx_s = _knn(src, jnp.transpose(src, (0, 2, 1)), 256, 0)
    idx_t = _knn(tgt, jnp.transpose(tgt, (0, 2, 1)), 256, N)
    idxf = jnp.concatenate([idx_s, idx_t], axis=1).reshape(TOT * KNB)
    agg = _sc_agg(a.reshape(TOT, D2), bc.reshape(TOT, D2), idxf)
    out = _final(fp.reshape(TOT, D), agg, W_node, b_node)
    return out.reshape(B, NM, D)


# speedup vs baseline: 12.7072x; 12.7072x over previous
import functools

import jax
import jax.numpy as jnp
from jax import lax
from jax.experimental import pallas as pl
from jax.experimental.pallas import tpu as pltpu
from jax.experimental.pallas import tpu_sc as plsc

B, N, M, D, KNB = 4, 2048, 512, 128, 6
NM = N + M
TOT = B * NM
D2 = 2 * D

_NC, _NS = 2, 16
_NW = _NC * _NS
_NPW = TOT // _NW
_CH = 16
_NCHUNK = _NPW // _CH


def _encode_body(x_ref, ws_ref, bs_ref, wt_ref, bt_ref, we_ref, be_ref,
                 wn_ref, a_ref, bc_ref, fp_ref):
    nb = pl.program_id(1)
    is_src = nb < (N // 512)
    w = jnp.where(is_src, ws_ref[...], wt_ref[...])
    bias = jnp.where(is_src, bs_ref[...], bt_ref[...])
    x = x_ref[0]
    emb = (x[:, 0][:, None] * w[0][None, :]
           + x[:, 1][:, None] * w[1][None, :]
           + x[:, 2][:, None] * w[2][None, :])
    emb = jnp.maximum(emb + bias, 0.0)
    a_ref[0] = jnp.dot(emb, we_ref[:D, :],
                       preferred_element_type=jnp.float32)
    bc_ref[0] = (jnp.dot(emb, we_ref[D:, :],
                         preferred_element_type=jnp.float32)
                 + be_ref[...])
    fp_ref[0] = jnp.dot(emb, wn_ref[:D, :],
                        preferred_element_type=jnp.float32)


def _encode(xyz, W_src, b_src, W_tgt, b_tgt, W_edge, b_edge, W_node):
    nblk = NM // 512
    full = lambda shape: pl.BlockSpec(shape, lambda b, n: (0,) * len(shape))
    return pl.pallas_call(
        _encode_body,
        grid=(B, nblk),
        in_specs=[
            pl.BlockSpec((1, 512, 3), lambda b, n: (b, n, 0)),
            full((3, D)), full((1, D)), full((3, D)), full((1, D)),
            full((D2, D2)), full((1, D2)), full((3 * D, D)),
        ],
        out_specs=[
            pl.BlockSpec((1, 512, D2), lambda b, n: (b, n, 0)),
            pl.BlockSpec((1, 512, D2), lambda b, n: (b, n, 0)),
            pl.BlockSpec((1, 512, D), lambda b, n: (b, n, 0)),
        ],
        out_shape=[
            jax.ShapeDtypeStruct((B, NM, D2), jnp.float32),
            jax.ShapeDtypeStruct((B, NM, D2), jnp.float32),
            jax.ShapeDtypeStruct((B, NM, D), jnp.float32),
        ],
    )(xyz, W_src, b_src.reshape(1, D), W_tgt, b_tgt.reshape(1, D),
      W_edge, b_edge.reshape(1, D2), W_node)


def _knn_body(xr_ref, xt_ref, out_ref, *, rows, cols, goff):
    b = pl.program_id(0)
    rb = pl.program_id(1)
    xr = xr_ref[0]
    xt = xt_ref[0]
    d = None
    for c in range(3):
        diff = xr[:, c][:, None] - xt[c][None, :]
        sq = diff * diff
        d = sq if d is None else d + sq
    colf = lax.broadcasted_iota(jnp.int32, (rows, cols), 1).astype(jnp.float32)
    rowf = (lax.broadcasted_iota(jnp.int32, (rows, cols), 0)
            + rb * rows).astype(jnp.float32)
    d = jnp.where(colf == rowf, jnp.float32(1e10), d)
    sent = jnp.float32(2 * cols)
    big = jnp.float32(3e38)
    js = []
    for k in range(KNB):
        m = jnp.min(d, axis=1, keepdims=True)
        cand = jnp.where(d == m, colf, sent)
        j = jnp.min(cand, axis=1, keepdims=True)
        js.append(j)
        if k < KNB - 1:
            d = jnp.where(cand == j, big, d)
    idx = jnp.concatenate(js, axis=1).astype(jnp.int32)
    out_ref[0] = idx + (b * NM + goff)


def _knn(xyz, xyzT, rows, goff):
    npts = xyz.shape[1]
    nblk = npts // rows
    body = functools.partial(_knn_body, rows=rows, cols=npts, goff=goff)
    return pl.pallas_call(
        body,
        grid=(B, nblk),
        in_specs=[
            pl.BlockSpec((1, rows, 3), lambda b, n: (b, n, 0)),
            pl.BlockSpec((1, 3, npts), lambda b, n: (b, 0, 0)),
        ],
        out_specs=pl.BlockSpec((1, rows, KNB), lambda b, n: (b, n, 0)),
        out_shape=jax.ShapeDtypeStruct((B, npts, KNB), jnp.int32),
    )(xyz, xyzT)


def _sc_agg_body(a_hbm, bc_hbm, idx_hbm, out_hbm, idx_v, rows_v, bc_v,
                 acc_v, sem):
    wid = lax.axis_index("s") * _NC + lax.axis_index("c")
    base0 = wid * _NPW

    def chunk_body(ch, carry):
        nb = base0 + ch * _CH
        pltpu.sync_copy(idx_hbm.at[pl.ds(nb * KNB, _CH * KNB)], idx_v)
        cp = pltpu.async_copy(a_hbm.at[idx_v], rows_v, sem)
        pltpu.sync_copy(bc_hbm.at[pl.ds(nb, _CH)], bc_v)
        cp.wait()

        def node_body(i, c2):
            for c in range(D2 // 16):
                sl = pl.ds(c * 16, 16)
                bcv = bc_v[i, sl]
                acc = jnp.maximum(rows_v[i * KNB, sl] + bcv, 0.0)
                for k in range(1, KNB):
                    acc = acc + jnp.maximum(rows_v[i * KNB + k, sl] + bcv,
                                            0.0)
                acc_v[i, sl] = acc
            return c2

        lax.fori_loop(0, _CH, node_body, 0)
        pltpu.sync_copy(acc_v, out_hbm.at[pl.ds(nb, _CH)])
        return carry

    lax.fori_loop(0, _NCHUNK, chunk_body, 0)


def _sc_agg(a2, bc2, idxf):
    mesh = plsc.VectorSubcoreMesh(core_axis_name="c", subcore_axis_name="s")
    fn = pl.kernel(
        _sc_agg_body,
        out_type=jax.ShapeDtypeStruct((TOT, D2), jnp.float32),
        mesh=mesh,
        scratch_types=[
            pltpu.VMEM((_CH * KNB,), jnp.int32),
            pltpu.VMEM((_CH * KNB, D2), jnp.float32),
            pltpu.VMEM((_CH, D2), jnp.float32),
            pltpu.VMEM((_CH, D2), jnp.float32),
            pltpu.SemaphoreType.DMA,
        ],
    )
    return fn(a2, bc2, idxf)


def _final_body(fp_ref, agg_ref, wn_ref, bn_ref, out_ref):
    out_ref[...] = jnp.maximum(
        fp_ref[...]
        + jnp.dot(agg_ref[...], wn_ref[...],
                  preferred_element_type=jnp.float32)
        + bn_ref[...], 0.0)


def _final(fp2, agg, W_node, b_node):
    nblk = TOT // 512
    return pl.pallas_call(
        _final_body,
        grid=(nblk,),
        in_specs=[
            pl.BlockSpec((512, D), lambda n: (n, 0)),
            pl.BlockSpec((512, D2), lambda n: (n, 0)),
            pl.BlockSpec((D2, D), lambda n: (0, 0)),
            pl.BlockSpec((1, D), lambda n: (0, 0)),
        ],
        out_specs=pl.BlockSpec((512, D), lambda n: (n, 0)),
        out_shape=jax.ShapeDtypeStruct((TOT, D), jnp.float32),
    )(fp2, agg, W_node[D:, :], b_node.reshape(1, D))


def kernel(src, tgt, W_src, b_src, W_tgt, b_tgt, W_edge, b_edge, W_node,
           b_node):
    xyz = jnp.concatenate([src, tgt], axis=1)
    a, bc, fp = _encode(xyz, W_src, b_src, W_tgt, b_tgt, W_edge, b_edge,
                        W_node)
    idx_s = _knn(src, jnp.transpose(src, (0, 2, 1)), 256, 0)
    idx_t = _knn(tgt, jnp.transpose(tgt, (0, 2, 1)), 256, N)
    idxf = jnp.concatenate([idx_s, idx_t], axis=1).reshape(TOT * KNB)
    agg = _sc_agg(a.reshape(TOT, D2), bc.reshape(TOT, D2), idxf)
    out = _final(fp.reshape(TOT, D), agg, W_node, b_node)
    return out.reshape(B, NM, D)


# speedup vs baseline: 13.2571x; 1.0433x over previous
import functools

import jax
import jax.numpy as jnp
from jax import lax
from jax.experimental import pallas as pl
from jax.experimental.pallas import tpu as pltpu
from jax.experimental.pallas import tpu_sc as plsc

B, N, M, D, KNB = 4, 2048, 512, 128, 6
NM = N + M
TOT = B * NM
D2 = 2 * D

_NC, _NS = 2, 16
_NW = _NC * _NS
_NPW = TOT // _NW
_CH = 16
_NCHUNK = _NPW // _CH


def _encode_body(x_ref, ws_ref, bs_ref, wt_ref, bt_ref, we_ref, be_ref,
                 wn_ref, a_ref, bc_ref, fp_ref):
    nb = pl.program_id(1)
    is_src = nb < (N // 512)
    w = jnp.where(is_src, ws_ref[...], wt_ref[...])
    bias = jnp.where(is_src, bs_ref[...], bt_ref[...])
    x = x_ref[0]
    emb = (x[:, 0][:, None] * w[0][None, :]
           + x[:, 1][:, None] * w[1][None, :]
           + x[:, 2][:, None] * w[2][None, :])
    emb = jnp.maximum(emb + bias, 0.0)
    a_ref[0] = jnp.dot(emb, we_ref[:D, :],
                       preferred_element_type=jnp.float32)
    bc_ref[0] = (jnp.dot(emb, we_ref[D:, :],
                         preferred_element_type=jnp.float32)
                 + be_ref[...])
    fp_ref[0] = jnp.dot(emb, wn_ref[:D, :],
                        preferred_element_type=jnp.float32)


def _encode(xyz, W_src, b_src, W_tgt, b_tgt, W_edge, b_edge, W_node):
    nblk = NM // 512
    full = lambda shape: pl.BlockSpec(shape, lambda b, n: (0,) * len(shape))
    return pl.pallas_call(
        _encode_body,
        grid=(B, nblk),
        in_specs=[
            pl.BlockSpec((1, 512, 3), lambda b, n: (b, n, 0)),
            full((3, D)), full((1, D)), full((3, D)), full((1, D)),
            full((D2, D2)), full((1, D2)), full((3 * D, D)),
        ],
        out_specs=[
            pl.BlockSpec((1, 512, D2), lambda b, n: (b, n, 0)),
            pl.BlockSpec((1, 512, D2), lambda b, n: (b, n, 0)),
            pl.BlockSpec((1, 512, D), lambda b, n: (b, n, 0)),
        ],
        out_shape=[
            jax.ShapeDtypeStruct((B, NM, D2), jnp.float32),
            jax.ShapeDtypeStruct((B, NM, D2), jnp.float32),
            jax.ShapeDtypeStruct((B, NM, D), jnp.float32),
        ],
    )(xyz, W_src, b_src.reshape(1, D), W_tgt, b_tgt.reshape(1, D),
      W_edge, b_edge.reshape(1, D2), W_node)


def _knn_body(xr_ref, xt_ref, out_ref, *, rows, cols, goff):
    b = pl.program_id(0)
    rb = pl.program_id(1)
    xr = xr_ref[0]
    xt = xt_ref[0]
    d = None
    for c in range(3):
        diff = xr[:, c][:, None] - xt[c][None, :]
        sq = diff * diff
        d = sq if d is None else d + sq
    colf = lax.broadcasted_iota(jnp.int32, (rows, cols), 1).astype(jnp.float32)
    rowf = (lax.broadcasted_iota(jnp.int32, (rows, cols), 0)
            + rb * rows).astype(jnp.float32)
    d = jnp.where(colf == rowf, jnp.float32(1e10), d)
    sent = jnp.float32(2 * cols)
    big = jnp.float32(3e38)
    js = []
    for k in range(KNB):
        m = jnp.min(d, axis=1, keepdims=True)
        cand = jnp.where(d == m, colf, sent)
        j = jnp.min(cand, axis=1, keepdims=True)
        js.append(j)
        if k < KNB - 1:
            d = jnp.where(cand == j, big, d)
    idx = jnp.concatenate(js, axis=1).astype(jnp.int32)
    out_ref[0] = idx + (b * NM + goff)


def _knn(xyz, xyzT, rows, goff):
    npts = xyz.shape[1]
    nblk = npts // rows
    body = functools.partial(_knn_body, rows=rows, cols=npts, goff=goff)
    return pl.pallas_call(
        body,
        grid=(B, nblk),
        in_specs=[
            pl.BlockSpec((1, rows, 3), lambda b, n: (b, n, 0)),
            pl.BlockSpec((1, 3, npts), lambda b, n: (b, 0, 0)),
        ],
        out_specs=pl.BlockSpec((1, rows, KNB), lambda b, n: (b, n, 0)),
        out_shape=jax.ShapeDtypeStruct((B, npts, KNB), jnp.int32),
    )(xyz, xyzT)


def _sc_agg_body(a_hbm, bc_hbm, idx_hbm, out_hbm, idx_v, rows_v, bc_v,
                 acc_v, sem0, sem1):
    wid = lax.axis_index("s") * _NC + lax.axis_index("c")
    base0 = wid * _NPW
    sems = (sem0, sem1)

    def start(ch, buf):
        nb = base0 + ch * _CH
        pltpu.sync_copy(idx_hbm.at[pl.ds(nb * KNB, _CH * KNB)],
                        idx_v.at[buf])
        pltpu.async_copy(a_hbm.at[idx_v.at[buf]], rows_v.at[buf], sems[buf])
        pltpu.sync_copy(bc_hbm.at[pl.ds(nb, _CH)], bc_v.at[buf])

    def wait(buf):
        pltpu.make_async_copy(a_hbm.at[idx_v.at[buf]], rows_v.at[buf],
                              sems[buf]).wait()

    def compute(ch, buf):
        nb = base0 + ch * _CH
        r_v, b_v = rows_v.at[buf], bc_v.at[buf]

        def node_body(i, c2):
            for c in range(D2 // 16):
                sl = pl.ds(c * 16, 16)
                bcv = b_v[i, sl]
                acc = jnp.maximum(r_v[i * KNB, sl] + bcv, 0.0)
                for k in range(1, KNB):
                    acc = acc + jnp.maximum(r_v[i * KNB + k, sl] + bcv, 0.0)
                acc_v[i, sl] = acc
            return c2

        lax.fori_loop(0, _CH, node_body, 0)
        pltpu.sync_copy(acc_v, out_hbm.at[pl.ds(nb, _CH)])

    start(0, 0)

    def pair_body(p, carry):
        for b in range(2):
            ch = p * 2 + b

            @pl.when(ch + 1 < _NCHUNK)
            def _():
                start(ch + 1, 1 - b)

            wait(b)
            compute(ch, b)
        return carry

    lax.fori_loop(0, _NCHUNK // 2, pair_body, 0)


def _sc_agg(a2, bc2, idxf):
    mesh = plsc.VectorSubcoreMesh(core_axis_name="c", subcore_axis_name="s")
    fn = pl.kernel(
        _sc_agg_body,
        out_type=jax.ShapeDtypeStruct((TOT, D2), jnp.float32),
        mesh=mesh,
        scratch_types=[
            pltpu.VMEM((2, _CH * KNB), jnp.int32),
            pltpu.VMEM((2, _CH * KNB, D2), jnp.float32),
            pltpu.VMEM((2, _CH, D2), jnp.float32),
            pltpu.VMEM((_CH, D2), jnp.float32),
            pltpu.SemaphoreType.DMA,
            pltpu.SemaphoreType.DMA,
        ],
    )
    return fn(a2, bc2, idxf)


def _final_body(fp_ref, agg_ref, wn_ref, bn_ref, out_ref):
    out_ref[...] = jnp.maximum(
        fp_ref[...]
        + jnp.dot(agg_ref[...], wn_ref[...],
                  preferred_element_type=jnp.float32)
        + bn_ref[...], 0.0)


def _final(fp2, agg, W_node, b_node):
    nblk = TOT // 512
    return pl.pallas_call(
        _final_body,
        grid=(nblk,),
        in_specs=[
            pl.BlockSpec((512, D), lambda n: (n, 0)),
            pl.BlockSpec((512, D2), lambda n: (n, 0)),
            pl.BlockSpec((D2, D), lambda n: (0, 0)),
            pl.BlockSpec((1, D), lambda n: (0, 0)),
        ],
        out_specs=pl.BlockSpec((512, D), lambda n: (n, 0)),
        out_shape=jax.ShapeDtypeStruct((TOT, D), jnp.float32),
    )(fp2, agg, W_node[D:, :], b_node.reshape(1, D))


def kernel(src, tgt, W_src, b_src, W_tgt, b_tgt, W_edge, b_edge, W_node,
           b_node):
    xyz = jnp.concatenate([src, tgt], axis=1)
    a, bc, fp = _encode(xyz, W_src, b_src, W_tgt, b_tgt, W_edge, b_edge,
                        W_node)
    idx_s = _knn(src, jnp.transpose(src, (0, 2, 1)), 256, 0)
    idx_t = _knn(tgt, jnp.transpose(tgt, (0, 2, 1)), 256, N)
    idxf = jnp.concatenate([idx_s, idx_t], axis=1).reshape(TOT * KNB)
    agg = _sc_agg(a.reshape(TOT, D2), bc.reshape(TOT, D2), idxf)
    out = _final(fp.reshape(TOT, D), agg, W_node, b_node)
    return out.reshape(B, NM, D)


# speedup vs baseline: 14.1422x; 1.0668x over previous
import functools

import jax
import jax.numpy as jnp
from jax import lax
from jax.experimental import pallas as pl
from jax.experimental.pallas import tpu as pltpu
from jax.experimental.pallas import tpu_sc as plsc

B, N, M, D, KNB = 4, 2048, 512, 128, 6
NM = N + M
TOT = B * NM
D2 = 2 * D

_NC, _NS = 2, 16
_NW = _NC * _NS
_NPW = TOT // _NW
_CH = 16
_NCHUNK = _NPW // _CH


def _encode_body(x_ref, ws_ref, bs_ref, wt_ref, bt_ref, we_ref, be_ref,
                 wn_ref, a_ref, bc_ref, fp_ref):
    nb = pl.program_id(1)
    is_src = nb < (N // 512)
    w = jnp.where(is_src, ws_ref[...], wt_ref[...])
    bias = jnp.where(is_src, bs_ref[...], bt_ref[...])
    x = x_ref[0]
    emb = (x[:, 0][:, None] * w[0][None, :]
           + x[:, 1][:, None] * w[1][None, :]
           + x[:, 2][:, None] * w[2][None, :])
    emb = jnp.maximum(emb + bias, 0.0)
    a_ref[0] = jnp.dot(emb, we_ref[:D, :],
                       preferred_element_type=jnp.float32)
    bc_ref[0] = (jnp.dot(emb, we_ref[D:, :],
                         preferred_element_type=jnp.float32)
                 + be_ref[...])
    fp_ref[0] = jnp.dot(emb, wn_ref[:D, :],
                        preferred_element_type=jnp.float32)


def _encode(xyz, W_src, b_src, W_tgt, b_tgt, W_edge, b_edge, W_node):
    nblk = NM // 512
    full = lambda shape: pl.BlockSpec(shape, lambda b, n: (0,) * len(shape))
    return pl.pallas_call(
        _encode_body,
        grid=(B, nblk),
        in_specs=[
            pl.BlockSpec((1, 512, 3), lambda b, n: (b, n, 0)),
            full((3, D)), full((1, D)), full((3, D)), full((1, D)),
            full((D2, D2)), full((1, D2)), full((3 * D, D)),
        ],
        out_specs=[
            pl.BlockSpec((1, 512, D2), lambda b, n: (b, n, 0)),
            pl.BlockSpec((1, 512, D2), lambda b, n: (b, n, 0)),
            pl.BlockSpec((1, 512, D), lambda b, n: (b, n, 0)),
        ],
        out_shape=[
            jax.ShapeDtypeStruct((B, NM, D2), jnp.float32),
            jax.ShapeDtypeStruct((B, NM, D2), jnp.float32),
            jax.ShapeDtypeStruct((B, NM, D), jnp.float32),
        ],
    )(xyz, W_src, b_src.reshape(1, D), W_tgt, b_tgt.reshape(1, D),
      W_edge, b_edge.reshape(1, D2), W_node)


def _knn_body(xr_ref, xt_ref, out_ref, *, rows, cols, goff):
    b = pl.program_id(0)
    rb = pl.program_id(1)
    xr = xr_ref[0]
    xt = xt_ref[0]
    d = None
    for c in range(3):
        diff = xr[:, c][:, None] - xt[c][None, :]
        sq = diff * diff
        d = sq if d is None else d + sq
    colf = lax.broadcasted_iota(jnp.int32, (rows, cols), 1).astype(jnp.float32)
    rowf = (lax.broadcasted_iota(jnp.int32, (rows, cols), 0)
            + rb * rows).astype(jnp.float32)
    d = jnp.where(colf == rowf, jnp.float32(1e10), d)
    sent = jnp.float32(2 * cols)
    big = jnp.float32(3e38)
    js = []
    for k in range(KNB):
        m = jnp.min(d, axis=1, keepdims=True)
        cand = jnp.where(d == m, colf, sent)
        j = jnp.min(cand, axis=1, keepdims=True)
        js.append(j)
        if k < KNB - 1:
            d = jnp.where(cand == j, big, d)
    idx = jnp.concatenate(js, axis=1).astype(jnp.int32)
    out_ref[0] = idx + (b * NM + goff)


def _knn(xyz, xyzT, rows, goff):
    npts = xyz.shape[1]
    nblk = npts // rows
    body = functools.partial(_knn_body, rows=rows, cols=npts, goff=goff)
    return pl.pallas_call(
        body,
        grid=(B, nblk),
        in_specs=[
            pl.BlockSpec((1, rows, 3), lambda b, n: (b, n, 0)),
            pl.BlockSpec((1, 3, npts), lambda b, n: (b, 0, 0)),
        ],
        out_specs=pl.BlockSpec((1, rows, KNB), lambda b, n: (b, n, 0)),
        out_shape=jax.ShapeDtypeStruct((B, npts, KNB), jnp.int32),
    )(xyz, xyzT)


def _sc_agg_body(a_hbm, bc_hbm, idx_hbm, out_hbm, idx_v, rows_v, bc_v,
                 acc_v, sem0, sem1):
    wid = lax.axis_index("s") * _NC + lax.axis_index("c")
    base0 = wid * _NPW
    sems = (sem0, sem1)

    def start(ch, buf):
        nb = base0 + ch * _CH
        pltpu.sync_copy(idx_hbm.at[pl.ds(nb * KNB, _CH * KNB)],
                        idx_v.at[buf])
        pltpu.async_copy(a_hbm.at[idx_v.at[buf]], rows_v.at[buf], sems[buf])
        pltpu.sync_copy(bc_hbm.at[pl.ds(nb, _CH)], bc_v.at[buf])

    def wait(buf):
        pltpu.make_async_copy(a_hbm.at[idx_v.at[buf]], rows_v.at[buf],
                              sems[buf]).wait()

    def compute(ch, buf):
        nb = base0 + ch * _CH
        r_v, b_v = rows_v.at[buf], bc_v.at[buf]

        @plsc.parallel_loop(0, _CH, unroll=2)
        def node_body(i):
            for c in range(D2 // 16):
                sl = pl.ds(c * 16, 16)
                bcv = b_v[i, sl]
                acc = jnp.maximum(r_v[i * KNB, sl] + bcv, 0.0)
                for k in range(1, KNB):
                    acc = acc + jnp.maximum(r_v[i * KNB + k, sl] + bcv, 0.0)
                acc_v[i, sl] = acc
        pltpu.sync_copy(acc_v, out_hbm.at[pl.ds(nb, _CH)])

    start(0, 0)

    def pair_body(p, carry):
        for b in range(2):
            ch = p * 2 + b

            @pl.when(ch + 1 < _NCHUNK)
            def _():
                start(ch + 1, 1 - b)

            wait(b)
            compute(ch, b)
        return carry

    lax.fori_loop(0, _NCHUNK // 2, pair_body, 0)


def _sc_agg(a2, bc2, idxf):
    mesh = plsc.VectorSubcoreMesh(core_axis_name="c", subcore_axis_name="s")
    fn = pl.kernel(
        _sc_agg_body,
        out_type=jax.ShapeDtypeStruct((TOT, D2), jnp.float32),
        mesh=mesh,
        scratch_types=[
            pltpu.VMEM((2, _CH * KNB), jnp.int32),
            pltpu.VMEM((2, _CH * KNB, D2), jnp.float32),
            pltpu.VMEM((2, _CH, D2), jnp.float32),
            pltpu.VMEM((_CH, D2), jnp.float32),
            pltpu.SemaphoreType.DMA,
            pltpu.SemaphoreType.DMA,
        ],
    )
    return fn(a2, bc2, idxf)


def _final_body(fp_ref, agg_ref, wn_ref, bn_ref, out_ref):
    out_ref[...] = jnp.maximum(
        fp_ref[...]
        + jnp.dot(agg_ref[...], wn_ref[...],
                  preferred_element_type=jnp.float32)
        + bn_ref[...], 0.0)


def _final(fp2, agg, W_node, b_node):
    nblk = TOT // 512
    return pl.pallas_call(
        _final_body,
        grid=(nblk,),
        in_specs=[
            pl.BlockSpec((512, D), lambda n: (n, 0)),
            pl.BlockSpec((512, D2), lambda n: (n, 0)),
            pl.BlockSpec((D2, D), lambda n: (0, 0)),
            pl.BlockSpec((1, D), lambda n: (0, 0)),
        ],
        out_specs=pl.BlockSpec((512, D), lambda n: (n, 0)),
        out_shape=jax.ShapeDtypeStruct((TOT, D), jnp.float32),
    )(fp2, agg, W_node[D:, :], b_node.reshape(1, D))


def kernel(src, tgt, W_src, b_src, W_tgt, b_tgt, W_edge, b_edge, W_node,
           b_node):
    xyz = jnp.concatenate([src, tgt], axis=1)
    a, bc, fp = _encode(xyz, W_src, b_src, W_tgt, b_tgt, W_edge, b_edge,
                        W_node)
    idx_s = _knn(src, jnp.transpose(src, (0, 2, 1)), 256, 0)
    idx_t = _knn(tgt, jnp.transpose(tgt, (0, 2, 1)), 256, N)
    idxf = jnp.concatenate([idx_s, idx_t], axis=1).reshape(TOT * KNB)
    agg = _sc_agg(a.reshape(TOT, D2), bc.reshape(TOT, D2), idxf)
    out = _final(fp.reshape(TOT, D), agg, W_node, b_node)
    return out.reshape(B, NM, D)


# speedup vs baseline: 14.7841x; 1.0454x over previous
import functools

import jax
import jax.numpy as jnp
from jax import lax
from jax.experimental import pallas as pl
from jax.experimental.pallas import tpu as pltpu
from jax.experimental.pallas import tpu_sc as plsc

B, N, M, D, KNB = 4, 2048, 512, 128, 6
NM = N + M
TOT = B * NM
D2 = 2 * D

_NC, _NS = 2, 16
_NW = _NC * _NS
_NPW = TOT // _NW
_CH = 16
_NCHUNK = _NPW // _CH


def _encode_body(x_ref, ws_ref, bs_ref, wt_ref, bt_ref, we_ref, be_ref,
                 wn_ref, a_ref, bc_ref, fp_ref):
    nb = pl.program_id(1)
    is_src = nb < (N // 512)
    w = jnp.where(is_src, ws_ref[...], wt_ref[...])
    bias = jnp.where(is_src, bs_ref[...], bt_ref[...])
    x = x_ref[0]
    emb = (x[:, 0][:, None] * w[0][None, :]
           + x[:, 1][:, None] * w[1][None, :]
           + x[:, 2][:, None] * w[2][None, :])
    emb = jnp.maximum(emb + bias, 0.0)
    a_ref[0] = jnp.dot(emb, we_ref[:D, :],
                       preferred_element_type=jnp.float32)
    bc_ref[0] = (jnp.dot(emb, we_ref[D:, :],
                         preferred_element_type=jnp.float32)
                 + be_ref[...])
    fp_ref[0] = jnp.dot(emb, wn_ref[:D, :],
                        preferred_element_type=jnp.float32)


def _encode(xyz, W_src, b_src, W_tgt, b_tgt, W_edge, b_edge, W_node):
    nblk = NM // 512
    full = lambda shape: pl.BlockSpec(shape, lambda b, n: (0,) * len(shape))
    return pl.pallas_call(
        _encode_body,
        grid=(B, nblk),
        in_specs=[
            pl.BlockSpec((1, 512, 3), lambda b, n: (b, n, 0)),
            full((3, D)), full((1, D)), full((3, D)), full((1, D)),
            full((D2, D2)), full((1, D2)), full((3 * D, D)),
        ],
        out_specs=[
            pl.BlockSpec((1, 512, D2), lambda b, n: (b, n, 0)),
            pl.BlockSpec((1, 512, D2), lambda b, n: (b, n, 0)),
            pl.BlockSpec((1, 512, D), lambda b, n: (b, n, 0)),
        ],
        out_shape=[
            jax.ShapeDtypeStruct((B, NM, D2), jnp.float32),
            jax.ShapeDtypeStruct((B, NM, D2), jnp.float32),
            jax.ShapeDtypeStruct((B, NM, D), jnp.float32),
        ],
    )(xyz, W_src, b_src.reshape(1, D), W_tgt, b_tgt.reshape(1, D),
      W_edge, b_edge.reshape(1, D2), W_node)


def _knn_body(xr_ref, xt_ref, out_ref, *, rows, cols, goff):
    b = pl.program_id(0)
    rb = pl.program_id(1)
    xr = xr_ref[0]
    xt = xt_ref[0]
    d = None
    for c in range(3):
        diff = xr[:, c][:, None] - xt[c][None, :]
        sq = diff * diff
        d = sq if d is None else d + sq
    colf = lax.broadcasted_iota(jnp.int32, (rows, cols), 1).astype(jnp.float32)
    rowf = (lax.broadcasted_iota(jnp.int32, (rows, cols), 0)
            + rb * rows).astype(jnp.float32)
    d = jnp.where(colf == rowf, jnp.float32(1e10), d)
    sent = jnp.float32(2 * cols)
    big = jnp.float32(3e38)
    js = []
    for k in range(KNB):
        m = jnp.min(d, axis=1, keepdims=True)
        cand = jnp.where(d == m, colf, sent)
        j = jnp.min(cand, axis=1, keepdims=True)
        js.append(j)
        if k < KNB - 1:
            d = jnp.where(cand == j, big, d)
    idx = jnp.concatenate(js, axis=1).astype(jnp.int32)
    out_ref[0] = idx + (b * NM + goff)


def _knn(xyz, xyzT, rows, goff):
    npts = xyz.shape[1]
    nblk = npts // rows
    body = functools.partial(_knn_body, rows=rows, cols=npts, goff=goff)
    return pl.pallas_call(
        body,
        grid=(B, nblk),
        in_specs=[
            pl.BlockSpec((1, rows, 3), lambda b, n: (b, n, 0)),
            pl.BlockSpec((1, 3, npts), lambda b, n: (b, 0, 0)),
        ],
        out_specs=pl.BlockSpec((1, rows, KNB), lambda b, n: (b, n, 0)),
        out_shape=jax.ShapeDtypeStruct((B, npts, KNB), jnp.int32),
    )(xyz, xyzT)


def _sc_agg_body(a_hbm, bc_hbm, idx_hbm, out_hbm, idx_v, rows_v, bc_v,
                 acc_v, sem0, sem1):
    wid = lax.axis_index("s") * _NC + lax.axis_index("c")
    base0 = wid * _NPW
    sems = (sem0, sem1)

    def start(ch, buf):
        nb = base0 + ch * _CH
        pltpu.sync_copy(idx_hbm.at[pl.ds(nb * KNB, _CH * KNB)],
                        idx_v.at[buf])
        pltpu.async_copy(a_hbm.at[idx_v.at[buf]], rows_v.at[buf], sems[buf])
        pltpu.sync_copy(bc_hbm.at[pl.ds(nb, _CH)], bc_v.at[buf])

    def wait(buf):
        pltpu.make_async_copy(a_hbm.at[idx_v.at[buf]], rows_v.at[buf],
                              sems[buf]).wait()

    def compute(ch, buf):
        nb = base0 + ch * _CH
        r_v, b_v = rows_v.at[buf], bc_v.at[buf]

        @plsc.parallel_loop(0, _CH, unroll=4)
        def node_body(i):
            for c in range(D2 // 16):
                sl = pl.ds(c * 16, 16)
                bcv = b_v[i, sl]
                acc = jnp.maximum(r_v[i * KNB, sl] + bcv, 0.0)
                for k in range(1, KNB):
                    acc = acc + jnp.maximum(r_v[i * KNB + k, sl] + bcv, 0.0)
                acc_v[i, sl] = acc
        pltpu.sync_copy(acc_v, out_hbm.at[pl.ds(nb, _CH)])

    start(0, 0)

    def pair_body(p, carry):
        for b in range(2):
            ch = p * 2 + b

            @pl.when(ch + 1 < _NCHUNK)
            def _():
                start(ch + 1, 1 - b)

            wait(b)
            compute(ch, b)
        return carry

    lax.fori_loop(0, _NCHUNK // 2, pair_body, 0)


def _sc_agg(a2, bc2, idxf):
    mesh = plsc.VectorSubcoreMesh(core_axis_name="c", subcore_axis_name="s")
    fn = pl.kernel(
        _sc_agg_body,
        out_type=jax.ShapeDtypeStruct((TOT, D2), jnp.float32),
        mesh=mesh,
        scratch_types=[
            pltpu.VMEM((2, _CH * KNB), jnp.int32),
            pltpu.VMEM((2, _CH * KNB, D2), jnp.float32),
            pltpu.VMEM((2, _CH, D2), jnp.float32),
            pltpu.VMEM((_CH, D2), jnp.float32),
            pltpu.SemaphoreType.DMA,
            pltpu.SemaphoreType.DMA,
        ],
    )
    return fn(a2, bc2, idxf)


def _final_body(fp_ref, agg_ref, wn_ref, bn_ref, out_ref):
    out_ref[...] = jnp.maximum(
        fp_ref[...]
        + jnp.dot(agg_ref[...], wn_ref[...],
                  preferred_element_type=jnp.float32)
        + bn_ref[...], 0.0)


def _final(fp2, agg, W_node, b_node):
    nblk = TOT // 512
    return pl.pallas_call(
        _final_body,
        grid=(nblk,),
        in_specs=[
            pl.BlockSpec((512, D), lambda n: (n, 0)),
            pl.BlockSpec((512, D2), lambda n: (n, 0)),
            pl.BlockSpec((D2, D), lambda n: (0, 0)),
            pl.BlockSpec((1, D), lambda n: (0, 0)),
        ],
        out_specs=pl.BlockSpec((512, D), lambda n: (n, 0)),
        out_shape=jax.ShapeDtypeStruct((TOT, D), jnp.float32),
    )(fp2, agg, W_node[D:, :], b_node.reshape(1, D))


def kernel(src, tgt, W_src, b_src, W_tgt, b_tgt, W_edge, b_edge, W_node,
           b_node):
    xyz = jnp.concatenate([src, tgt], axis=1)
    a, bc, fp = _encode(xyz, W_src, b_src, W_tgt, b_tgt, W_edge, b_edge,
                        W_node)
    idx_s = _knn(src, jnp.transpose(src, (0, 2, 1)), 256, 0)
    idx_t = _knn(tgt, jnp.transpose(tgt, (0, 2, 1)), 256, N)
    idxf = jnp.concatenate([idx_s, idx_t], axis=1).reshape(TOT * KNB)
    agg = _sc_agg(a.reshape(TOT, D2), bc.reshape(TOT, D2), idxf)
    out = _final(fp.reshape(TOT, D), agg, W_node, b_node)
    return out.reshape(B, NM, D)


# speedup vs baseline: 14.9909x; 1.0140x over previous
import functools

import jax
import jax.numpy as jnp
from jax import lax
from jax.experimental import pallas as pl
from jax.experimental.pallas import tpu as pltpu
from jax.experimental.pallas import tpu_sc as plsc

B, N, M, D, KNB = 4, 2048, 512, 128, 6
NM = N + M
TOT = B * NM
D2 = 2 * D

_NC, _NS = 2, 16
_NW = _NC * _NS
_NPW = TOT // _NW
_CH = 16
_NCHUNK = _NPW // _CH


def _encode_body(x_ref, ws_ref, bs_ref, wt_ref, bt_ref, we_ref, be_ref,
                 wn_ref, a_ref, bc_ref, fp_ref):
    nb = pl.program_id(1)
    is_src = nb < (N // 512)
    w = jnp.where(is_src, ws_ref[...], wt_ref[...])
    bias = jnp.where(is_src, bs_ref[...], bt_ref[...])
    x = x_ref[0]
    emb = (x[:, 0][:, None] * w[0][None, :]
           + x[:, 1][:, None] * w[1][None, :]
           + x[:, 2][:, None] * w[2][None, :])
    emb = jnp.maximum(emb + bias, 0.0)
    a_ref[0] = jnp.dot(emb, we_ref[:D, :],
                       preferred_element_type=jnp.float32)
    bc_ref[0] = (jnp.dot(emb, we_ref[D:, :],
                         preferred_element_type=jnp.float32)
                 + be_ref[...])
    fp_ref[0] = jnp.dot(emb, wn_ref[:D, :],
                        preferred_element_type=jnp.float32)


def _encode(xyz, W_src, b_src, W_tgt, b_tgt, W_edge, b_edge, W_node):
    nblk = NM // 512
    full = lambda shape: pl.BlockSpec(shape, lambda b, n: (0,) * len(shape))
    return pl.pallas_call(
        _encode_body,
        grid=(B, nblk),
        in_specs=[
            pl.BlockSpec((1, 512, 3), lambda b, n: (b, n, 0)),
            full((3, D)), full((1, D)), full((3, D)), full((1, D)),
            full((D2, D2)), full((1, D2)), full((3 * D, D)),
        ],
        out_specs=[
            pl.BlockSpec((1, 512, D2), lambda b, n: (b, n, 0)),
            pl.BlockSpec((1, 512, D2), lambda b, n: (b, n, 0)),
            pl.BlockSpec((1, 512, D), lambda b, n: (b, n, 0)),
        ],
        out_shape=[
            jax.ShapeDtypeStruct((B, NM, D2), jnp.float32),
            jax.ShapeDtypeStruct((B, NM, D2), jnp.float32),
            jax.ShapeDtypeStruct((B, NM, D), jnp.float32),
        ],
    )(xyz, W_src, b_src.reshape(1, D), W_tgt, b_tgt.reshape(1, D),
      W_edge, b_edge.reshape(1, D2), W_node)


def _knn_body(xr_ref, xt_ref, out_ref, *, rows, cols, goff):
    b = pl.program_id(0)
    rb = pl.program_id(1)
    xr = xr_ref[0]
    xt = xt_ref[0]
    d = None
    for c in range(3):
        diff = xr[:, c][:, None] - xt[c][None, :]
        sq = diff * diff
        d = sq if d is None else d + sq
    coli = lax.broadcasted_iota(jnp.int32, (rows, cols), 1)
    rowi = lax.broadcasted_iota(jnp.int32, (rows, cols), 0) + rb * rows
    d = jnp.where(coli == rowi, jnp.float32(1e10), d)
    big = jnp.float32(3e38)
    js = []
    for k in range(KNB):
        j = jnp.argmin(d, axis=1)[:, None]
        js.append(j)
        if k < KNB - 1:
            d = jnp.where(coli == j, big, d)
    idx = jnp.concatenate(js, axis=1).astype(jnp.int32)
    out_ref[0] = idx + (b * NM + goff)


def _knn(xyz, xyzT, rows, goff):
    npts = xyz.shape[1]
    nblk = npts // rows
    body = functools.partial(_knn_body, rows=rows, cols=npts, goff=goff)
    return pl.pallas_call(
        body,
        grid=(B, nblk),
        in_specs=[
            pl.BlockSpec((1, rows, 3), lambda b, n: (b, n, 0)),
            pl.BlockSpec((1, 3, npts), lambda b, n: (b, 0, 0)),
        ],
        out_specs=pl.BlockSpec((1, rows, KNB), lambda b, n: (b, n, 0)),
        out_shape=jax.ShapeDtypeStruct((B, npts, KNB), jnp.int32),
    )(xyz, xyzT)


def _sc_agg_body(a_hbm, bc_hbm, idx_hbm, out_hbm, idx_v, rows_v, bc_v,
                 acc_v, sem0, sem1):
    wid = lax.axis_index("s") * _NC + lax.axis_index("c")
    base0 = wid * _NPW
    sems = (sem0, sem1)

    def start(ch, buf):
        nb = base0 + ch * _CH
        pltpu.sync_copy(idx_hbm.at[pl.ds(nb * KNB, _CH * KNB)],
                        idx_v.at[buf])
        pltpu.async_copy(a_hbm.at[idx_v.at[buf]], rows_v.at[buf], sems[buf])
        pltpu.sync_copy(bc_hbm.at[pl.ds(nb, _CH)], bc_v.at[buf])

    def wait(buf):
        pltpu.make_async_copy(a_hbm.at[idx_v.at[buf]], rows_v.at[buf],
                              sems[buf]).wait()

    def compute(ch, buf):
        nb = base0 + ch * _CH
        r_v, b_v = rows_v.at[buf], bc_v.at[buf]

        @plsc.parallel_loop(0, _CH, unroll=4)
        def node_body(i):
            for c in range(D2 // 16):
                sl = pl.ds(c * 16, 16)
                bcv = b_v[i, sl]
                acc = jnp.maximum(r_v[i * KNB, sl] + bcv, 0.0)
                for k in range(1, KNB):
                    acc = acc + jnp.maximum(r_v[i * KNB + k, sl] + bcv, 0.0)
                acc_v[i, sl] = acc
        pltpu.sync_copy(acc_v, out_hbm.at[pl.ds(nb, _CH)])

    start(0, 0)

    def pair_body(p, carry):
        for b in range(2):
            ch = p * 2 + b

            @pl.when(ch + 1 < _NCHUNK)
            def _():
                start(ch + 1, 1 - b)

            wait(b)
            compute(ch, b)
        return carry

    lax.fori_loop(0, _NCHUNK // 2, pair_body, 0)


def _sc_agg(a2, bc2, idxf):
    mesh = plsc.VectorSubcoreMesh(core_axis_name="c", subcore_axis_name="s")
    fn = pl.kernel(
        _sc_agg_body,
        out_type=jax.ShapeDtypeStruct((TOT, D2), jnp.float32),
        mesh=mesh,
        scratch_types=[
            pltpu.VMEM((2, _CH * KNB), jnp.int32),
            pltpu.VMEM((2, _CH * KNB, D2), jnp.float32),
            pltpu.VMEM((2, _CH, D2), jnp.float32),
            pltpu.VMEM((_CH, D2), jnp.float32),
            pltpu.SemaphoreType.DMA,
            pltpu.SemaphoreType.DMA,
        ],
    )
    return fn(a2, bc2, idxf)


def _final_body(fp_ref, agg_ref, wn_ref, bn_ref, out_ref):
    out_ref[...] = jnp.maximum(
        fp_ref[...]
        + jnp.dot(agg_ref[...], wn_ref[...],
                  preferred_element_type=jnp.float32)
        + bn_ref[...], 0.0)


def _final(fp2, agg, W_node, b_node):
    nblk = TOT // 512
    return pl.pallas_call(
        _final_body,
        grid=(nblk,),
        in_specs=[
            pl.BlockSpec((512, D), lambda n: (n, 0)),
            pl.BlockSpec((512, D2), lambda n: (n, 0)),
            pl.BlockSpec((D2, D), lambda n: (0, 0)),
            pl.BlockSpec((1, D), lambda n: (0, 0)),
        ],
        out_specs=pl.BlockSpec((512, D), lambda n: (n, 0)),
        out_shape=jax.ShapeDtypeStruct((TOT, D), jnp.float32),
    )(fp2, agg, W_node[D:, :], b_node.reshape(1, D))


def kernel(src, tgt, W_src, b_src, W_tgt, b_tgt, W_edge, b_edge, W_node,
           b_node):
    xyz = jnp.concatenate([src, tgt], axis=1)
    a, bc, fp = _encode(xyz, W_src, b_src, W_tgt, b_tgt, W_edge, b_edge,
                        W_node)
    idx_s = _knn(src, jnp.transpose(src, (0, 2, 1)), 256, 0)
    idx_t = _knn(tgt, jnp.transpose(tgt, (0, 2, 1)), 256, N)
    idxf = jnp.concatenate([idx_s, idx_t], axis=1).reshape(TOT * KNB)
    agg = _sc_agg(a.reshape(TOT, D2), bc.reshape(TOT, D2), idxf)
    out = _final(fp.reshape(TOT, D), agg, W_node, b_node)
    return out.reshape(B, NM, D)
